# baseline (speedup 1.0000x reference)
import functools

import jax
import jax.numpy as jnp
from jax import lax
from jax.experimental import pallas as pl
from jax.experimental.pallas import tpu as pltpu
from jax.experimental.pallas import tpu_sc as plsc

_CHUNK = 128
_G = 2
_NBUF = 3


def _sc_gather(table, idxp, *, batch, n_tokens, seq_tail, dim):
    info = plsc.get_sparse_core_info()
    nw = info.num_cores * info.num_subcores
    nb = batch // nw
    n_stage = nb // _G
    out_rows = n_tokens + seq_tail
    tail_a = _CHUNK
    tail_b = -(-(seq_tail - _CHUNK) // 8) * 8
    spill = tail_b - (seq_tail - _CHUNK)
    assert spill <= n_tokens, "spill rows must stay within the next prefix"
    mesh = plsc.VectorSubcoreMesh(core_axis_name="c", subcore_axis_name="s")

    @functools.partial(
        pl.kernel,
        mesh=mesh,
        compiler_params=pltpu.CompilerParams(use_tc_tiling_on_sc=False),
        out_type=jax.ShapeDtypeStruct((batch * out_rows, dim), jnp.float32),
        scratch_types=[
            pltpu.VMEM((_NBUF, _G * out_rows + spill, dim), jnp.float32),
            pltpu.VMEM((nb, 2, _CHUNK), jnp.int32),
            pltpu.SemaphoreType.DMA((_NBUF,)),
            pltpu.SemaphoreType.DMA((_NBUF,)),
        ],
    )
    def k(table_hbm, idx_hbm, out_hbm, bufs, idxv, gsem, osem):
        wid = lax.axis_index("s") * info.num_cores + lax.axis_index("c")
        b0 = wid * nb

        pltpu.sync_copy(idx_hbm.at[pl.ds(b0, nb)], idxv)

        def issue_stage(s):
            kb = s % _NBUF
            for g in range(_G):
                row = g * out_rows
                pltpu.async_copy(
                    table_hbm.at[idxv.at[_G * s + g, 0]],
                    bufs.at[kb, pl.ds(row + n_tokens, tail_a)],
                    gsem.at[kb],
                )
                pltpu.async_copy(
                    table_hbm.at[idxv.at[_G * s + g, 1, pl.ds(0, tail_b)]],
                    bufs.at[kb, pl.ds(row + n_tokens + tail_a, tail_b)],
                    gsem.at[kb],
                )

        def wait_gathers(kb):
            for g in range(_G):
                row = g * out_rows
                pltpu.make_async_copy(
                    table_hbm.at[pl.ds(0, tail_a)],
                    bufs.at[kb, pl.ds(row + n_tokens, tail_a)],
                    gsem.at[kb],
                ).wait()
                pltpu.make_async_copy(
                    table_hbm.at[pl.ds(0, tail_b)],
                    bufs.at[kb, pl.ds(row + n_tokens + tail_a, tail_b)],
                    gsem.at[kb],
                ).wait()

        def wait_out(kb):
            pltpu.make_async_copy(
                bufs.at[kb, pl.ds(0, _G * out_rows)],
                out_hbm.at[pl.ds(0, _G * out_rows)],
                osem.at[kb],
            ).wait()

        issue_stage(0)

        def body(s, carry):
            kb = s % _NBUF
            wait_gathers(kb)
            pltpu.async_copy(
                bufs.at[kb, pl.ds(0, _G * out_rows)],
                out_hbm.at[pl.ds((b0 + _G * s) * out_rows, _G * out_rows)],
                osem.at[kb],
            )

            @pl.when(s + 1 < n_stage)
            def _():
                @pl.when(s >= _NBUF - 1)
                def _():
                    wait_out((s + 1) % _NBUF)

                issue_stage(s + 1)

            return carry

        lax.fori_loop(0, n_stage, body, 0)
        for t in range(_NBUF - 1):
            wait_out((n_stage - 1 - t) % _NBUF)

    return k(table, idxp)


def _merge(flat, g1, g2, *, batch, n_tokens, seq_tail, dim):
    out_rows = n_tokens + seq_tail

    def body(g1_ref, g2_ref, f_ref, o_ref):
        a = g1_ref[...]
        b = g2_ref[...]
        o_ref[0, :n_tokens, :] = (a / (a + b))[0]
        o_ref[0, n_tokens:, :] = f_ref[n_tokens:, :]

    return pl.pallas_call(
        body,
        grid=(batch,),
        in_specs=[
            pl.BlockSpec((1, n_tokens, dim), lambda i: (i, 0, 0)),
            pl.BlockSpec((1, n_tokens, dim), lambda i: (i, 0, 0)),
            pl.BlockSpec((out_rows, dim), lambda i: (i, 0)),
        ],
        out_specs=pl.BlockSpec((1, out_rows, dim), lambda i: (i, 0, 0)),
        out_shape=jax.ShapeDtypeStruct((batch, out_rows, dim), jnp.float32),
    )(g1, g2, flat)


def kernel(tokens, table, alpha, beta):
    n_tokens = alpha.shape[0]
    batch, seq = tokens.shape
    dim = table.shape[1]
    seq_tail = seq - n_tokens

    key = jax.random.key(42)
    ka, kb = jax.random.split(key)
    g1 = jax.random.gamma(ka, alpha, shape=(batch,) + alpha.shape)
    g2 = jax.random.gamma(kb, beta, shape=(batch,) + beta.shape)

    tail = tokens[:, n_tokens:]
    pad = (-seq_tail) % _CHUNK
    idxp = jnp.pad(tail, ((0, 0), (0, pad))).reshape(batch, -1, _CHUNK)
    flat = _sc_gather(
        table,
        idxp,
        batch=batch,
        n_tokens=n_tokens,
        seq_tail=seq_tail,
        dim=dim,
    )
    return _merge(
        flat, g1, g2, batch=batch, n_tokens=n_tokens, seq_tail=seq_tail, dim=dim
    )


# speedup vs baseline: 1.1608x; 1.1608x over previous
import functools

import jax
import jax.numpy as jnp
from jax import lax
from jax.experimental import pallas as pl
from jax.experimental.pallas import tpu as pltpu
from jax.experimental.pallas import tpu_sc as plsc

_CHUNK = 128
_G = 2
_NBUF = 3


def _sc_gather(table, idxp, *, batch, n_tokens, seq_tail, dim):
    info = plsc.get_sparse_core_info()
    nw = info.num_cores * info.num_subcores
    nb = batch // nw
    n_stage = nb // _G
    out_rows = n_tokens + seq_tail
    tail_a = _CHUNK
    tail_b = -(-(seq_tail - _CHUNK) // 8) * 8
    spill = tail_b - (seq_tail - _CHUNK)
    assert spill <= n_tokens, "spill rows must stay within the next prefix"
    mesh = plsc.VectorSubcoreMesh(core_axis_name="c", subcore_axis_name="s")

    @functools.partial(
        pl.kernel,
        mesh=mesh,
        compiler_params=pltpu.CompilerParams(use_tc_tiling_on_sc=False),
        out_type=jax.ShapeDtypeStruct((batch * out_rows, dim), jnp.float32),
        scratch_types=[
            pltpu.VMEM((_NBUF, _G * out_rows + spill, dim), jnp.float32),
            pltpu.VMEM((nb, 2, _CHUNK), jnp.int32),
            pltpu.SemaphoreType.DMA((_NBUF,)),
            pltpu.SemaphoreType.DMA((_NBUF,)),
        ],
    )
    def k(table_hbm, idx_hbm, out_hbm, bufs, idxv, gsem, osem):
        wid = lax.axis_index("s") * info.num_cores + lax.axis_index("c")
        b0 = wid * nb

        pltpu.sync_copy(idx_hbm.at[pl.ds(b0, nb)], idxv)

        def issue_stage(s):
            kb = s % _NBUF
            for g in range(_G):
                row = g * out_rows
                pltpu.async_copy(
                    table_hbm.at[idxv.at[_G * s + g, 0]],
                    bufs.at[kb, pl.ds(row + n_tokens, tail_a)],
                    gsem.at[kb],
                )
                pltpu.async_copy(
                    table_hbm.at[idxv.at[_G * s + g, 1, pl.ds(0, tail_b)]],
                    bufs.at[kb, pl.ds(row + n_tokens + tail_a, tail_b)],
                    gsem.at[kb],
                )

        def wait_gathers(kb):
            for g in range(_G):
                row = g * out_rows
                pltpu.make_async_copy(
                    table_hbm.at[pl.ds(0, tail_a)],
                    bufs.at[kb, pl.ds(row + n_tokens, tail_a)],
                    gsem.at[kb],
                ).wait()
                pltpu.make_async_copy(
                    table_hbm.at[pl.ds(0, tail_b)],
                    bufs.at[kb, pl.ds(row + n_tokens + tail_a, tail_b)],
                    gsem.at[kb],
                ).wait()

        def wait_out(kb):
            pltpu.make_async_copy(
                bufs.at[kb, pl.ds(0, _G * out_rows)],
                out_hbm.at[pl.ds(0, _G * out_rows)],
                osem.at[kb],
            ).wait()

        issue_stage(0)

        def body(s, carry):
            kb = s % _NBUF
            wait_gathers(kb)
            pltpu.async_copy(
                bufs.at[kb, pl.ds(0, _G * out_rows)],
                out_hbm.at[pl.ds((b0 + _G * s) * out_rows, _G * out_rows)],
                osem.at[kb],
            )

            @pl.when(s + 1 < n_stage)
            def _():
                @pl.when(s >= _NBUF - 1)
                def _():
                    wait_out((s + 1) % _NBUF)

                issue_stage(s + 1)

            return carry

        lax.fori_loop(0, n_stage, body, 0)
        for t in range(_NBUF - 1):
            wait_out((n_stage - 1 - t) % _NBUF)

    return k(table, idxp)


def _merge(flat3, g1, g2, *, batch, n_tokens, seq_tail, dim):
    out_rows = n_tokens + seq_tail
    blk = 8

    def body(g1_ref, g2_ref, f_ref, o_ref):
        a = g1_ref[...]
        b = g2_ref[...]
        o_ref[:, :n_tokens, :] = a / (a + b)
        o_ref[:, n_tokens:, :] = f_ref[:, n_tokens:, :]

    return pl.pallas_call(
        body,
        grid=(batch // blk,),
        in_specs=[
            pl.BlockSpec((blk, n_tokens, dim), lambda i: (i, 0, 0)),
            pl.BlockSpec((blk, n_tokens, dim), lambda i: (i, 0, 0)),
            pl.BlockSpec((blk, out_rows, dim), lambda i: (i, 0, 0)),
        ],
        out_specs=pl.BlockSpec((blk, out_rows, dim), lambda i: (i, 0, 0)),
        out_shape=jax.ShapeDtypeStruct((batch, out_rows, dim), jnp.float32),
    )(g1, g2, flat3)


def kernel(tokens, table, alpha, beta):
    n_tokens = alpha.shape[0]
    batch, seq = tokens.shape
    dim = table.shape[1]
    seq_tail = seq - n_tokens

    key = jax.random.key(42)
    ka, kb = jax.random.split(key)
    g1 = jax.random.gamma(ka, alpha, shape=(batch,) + alpha.shape)
    g2 = jax.random.gamma(kb, beta, shape=(batch,) + beta.shape)

    tail = tokens[:, n_tokens:]
    pad = (-seq_tail) % _CHUNK
    idxp = jnp.pad(tail, ((0, 0), (0, pad))).reshape(batch, -1, _CHUNK)
    flat = _sc_gather(
        table,
        idxp,
        batch=batch,
        n_tokens=n_tokens,
        seq_tail=seq_tail,
        dim=dim,
    )
    flat3 = flat.reshape(batch, n_tokens + seq_tail, dim)
    return _merge(
        flat3, g1, g2, batch=batch, n_tokens=n_tokens, seq_tail=seq_tail, dim=dim
    )


# speedup vs baseline: 3.0056x; 2.5893x over previous
import functools

import numpy as np

import jax
import jax.numpy as jnp
from jax import lax
from jax.experimental import pallas as pl
from jax.experimental.pallas import tpu as pltpu
from jax.experimental.pallas import tpu_sc as plsc

_CHUNK = 128
_G = 2
_NBUF = 3

_LANES_TC = 1024
_ROWS_TC = 8

_ONE_THIRD = np.float32(1.0 / 3.0)
_SQUEEZE = np.float32(0.0331)
_SQRT2 = np.float32(np.sqrt(2.0))
_NLO = np.float32(np.nextafter(np.float32(-1.0), np.float32(0.0)))


def _threefry2x32(k1, k2, x0, x1):
    ks = (k1, k2, k1 ^ k2 ^ np.uint32(0x1BD11BDA))
    rots = ((13, 15, 26, 6), (17, 29, 16, 24))
    x0 = x0 + k1
    x1 = x1 + k2
    for p in range(5):
        for r in rots[p % 2]:
            x0 = x0 + x1
            x1 = (x1 << r) | (x1 >> (32 - r))
            x1 = x0 ^ x1
        x0 = x0 + ks[(p + 1) % 3]
        x1 = x1 + ks[(p + 2) % 3] + np.uint32(p + 1)
    return x0, x1


def _bits_to_unit(bits):
    fb = (bits >> np.uint32(9)) | np.uint32(0x3F800000)
    return lax.bitcast_convert_type(fb, jnp.float32) - np.float32(1.0)


def _beta_prefix_body(kref, aref, oref):
    j = pl.program_id(0)
    shp = (2, _ROWS_TC, _LANES_TC)
    zero_u = jnp.zeros(shp, jnp.uint32)
    one_u = jnp.full(shp, 1, jnp.uint32)
    two_u = jnp.full(shp, 2, jnp.uint32)

    h = lax.broadcasted_iota(jnp.int32, shp, 0)
    bk1 = jnp.where(h == 0, kref[0, 0], kref[1, 0]).astype(jnp.uint32)
    bk2 = jnp.where(h == 0, kref[0, 1], kref[1, 1]).astype(jnp.uint32)

    r = lax.broadcasted_iota(jnp.int32, shp, 1)
    c = lax.broadcasted_iota(jnp.int32, shp, 2)
    i = ((j * _ROWS_TC + r) * _LANES_TC + c).astype(jnp.uint32)
    ek1, ek2 = _threefry2x32(bk1, bk2, zero_u, i)

    a_orig = aref[...]
    mask_ge1 = a_orig >= np.float32(1.0)
    alpha = jnp.where(mask_ge1, a_orig, a_orig + np.float32(1.0))
    d = alpha - _ONE_THIRD
    cc = _ONE_THIRD / lax.sqrt(d)

    key1, key2 = _threefry2x32(ek1, ek2, zero_u, zero_u)
    sk1, sk2 = _threefry2x32(ek1, ek2, zero_u, one_u)

    def lane_uniform(k1, k2):
        b1, b2 = _threefry2x32(k1, k2, zero_u, zero_u)
        return _bits_to_unit(b1 ^ b2)

    def outer_cond(carry):
        return jnp.any(carry[5] != 0)

    def outer_body(carry):
        k1, k2, X, V, U, act_i = carry
        act = act_i != 0
        nk1, nk2 = _threefry2x32(k1, k2, zero_u, zero_u)
        xk1, xk2 = _threefry2x32(k1, k2, zero_u, one_u)
        uk1, uk2 = _threefry2x32(k1, k2, zero_u, two_u)

        def inner_cond(s):
            return jnp.any(s[3] <= np.float32(0.0))

        def inner_body(s):
            a1, a2, x, v = s
            upd = v <= np.float32(0.0)
            na1, na2 = _threefry2x32(a1, a2, zero_u, zero_u)
            nb1, nb2 = _threefry2x32(a1, a2, zero_u, one_u)
            f = lane_uniform(nb1, nb2)
            u = jnp.maximum(_NLO, f * (np.float32(1.0) - _NLO) + _NLO)
            xn = _SQRT2 * lax.erf_inv(u)
            vn = np.float32(1.0) + xn * cc
            return (
                jnp.where(upd, na1, a1),
                jnp.where(upd, na2, a2),
                jnp.where(upd, xn, x),
                jnp.where(upd, vn, v),
            )

        _, _, x, v = lax.while_loop(
            inner_cond,
            inner_body,
            (xk1, xk2, jnp.zeros_like(X), jnp.full_like(V, -1.0)),
        )
        Xn = x * x
        Vn = (v * v) * v
        Un = jnp.maximum(np.float32(0.0), lane_uniform(uk1, uk2))
        rej = (Un >= np.float32(1.0) - _SQUEEZE * (Xn * Xn)) & (
            jnp.log(Un)
            >= Xn * np.float32(0.5) + d * ((np.float32(1.0) - Vn) + jnp.log(Vn))
        )
        return (
            jnp.where(act, nk1, k1),
            jnp.where(act, nk2, k2),
            jnp.where(act, Xn, X),
            jnp.where(act, Vn, V),
            jnp.where(act, Un, U),
            jnp.where(act & rej, 1, 0).astype(jnp.int32),
        )

    init = (
        key1,
        key2,
        jnp.zeros(shp, jnp.float32),
        jnp.ones(shp, jnp.float32),
        jnp.full(shp, 2.0, jnp.float32),
        jnp.ones(shp, jnp.int32),
    )
    _, _, _, V, _, _ = lax.while_loop(outer_cond, outer_body, init)

    us = jnp.maximum(np.float32(0.0), lane_uniform(sk1, sk2))
    boost = jnp.where(
        mask_ge1,
        np.float32(1.0),
        lax.pow(np.float32(1.0) - us, np.float32(1.0) / a_orig),
    )
    g = (d * V) * boost
    oref[...] = g[0] / (g[0] + g[1])


def _beta_prefix(keydata, params, *, n_el):
    n_rows = n_el // _LANES_TC
    grid = n_rows // _ROWS_TC
    return pl.pallas_call(
        _beta_prefix_body,
        grid=(grid,),
        in_specs=[
            pl.BlockSpec(memory_space=pltpu.MemorySpace.SMEM),
            pl.BlockSpec((2, _ROWS_TC, _LANES_TC), lambda j: (0, j, 0)),
        ],
        out_specs=pl.BlockSpec((_ROWS_TC, _LANES_TC), lambda j: (j, 0)),
        out_shape=jax.ShapeDtypeStruct((n_rows, _LANES_TC), jnp.float32),
    )(keydata, params)


def _sc_embed(table, idxp, pref, *, batch, n_tokens, seq_tail, dim):
    info = plsc.get_sparse_core_info()
    nw = info.num_cores * info.num_subcores
    nb = batch // nw
    n_stage = nb // _G
    n_pref = n_tokens * dim
    out_rows = n_tokens + seq_tail
    tail_a = _CHUNK
    tail_b = -(-(seq_tail - _CHUNK) // 8) * 8
    spill = tail_b - (seq_tail - _CHUNK)
    assert spill <= n_tokens, "spill rows must stay within the next prefix"
    per_row = dim // 16
    mesh = plsc.VectorSubcoreMesh(core_axis_name="c", subcore_axis_name="s")

    @functools.partial(
        pl.kernel,
        mesh=mesh,
        compiler_params=pltpu.CompilerParams(use_tc_tiling_on_sc=False),
        out_type=jax.ShapeDtypeStruct((batch * out_rows, dim), jnp.float32),
        scratch_types=[
            pltpu.VMEM((_NBUF, _G * out_rows + spill, dim), jnp.float32),
            pltpu.VMEM((nb, 2, _CHUNK), jnp.int32),
            pltpu.VMEM((nb, n_pref), jnp.float32),
            pltpu.SemaphoreType.DMA((_NBUF,)),
            pltpu.SemaphoreType.DMA((_NBUF,)),
        ],
    )
    def k(table_hbm, idx_hbm, pref_hbm, out_hbm, bufs, idxv, prefv, gsem, osem):
        wid = lax.axis_index("s") * info.num_cores + lax.axis_index("c")
        b0 = wid * nb

        pltpu.sync_copy(idx_hbm.at[pl.ds(b0, nb)], idxv)
        pltpu.sync_copy(pref_hbm.at[pl.ds(b0, nb)], prefv)

        def issue_stage(s):
            kb = s % _NBUF
            for g in range(_G):
                row = g * out_rows
                pltpu.async_copy(
                    table_hbm.at[idxv.at[_G * s + g, 0]],
                    bufs.at[kb, pl.ds(row + n_tokens, tail_a)],
                    gsem.at[kb],
                )
                pltpu.async_copy(
                    table_hbm.at[idxv.at[_G * s + g, 1, pl.ds(0, tail_b)]],
                    bufs.at[kb, pl.ds(row + n_tokens + tail_a, tail_b)],
                    gsem.at[kb],
                )

        def wait_gathers(kb):
            for g in range(_G):
                row = g * out_rows
                pltpu.make_async_copy(
                    table_hbm.at[pl.ds(0, tail_a)],
                    bufs.at[kb, pl.ds(row + n_tokens, tail_a)],
                    gsem.at[kb],
                ).wait()
                pltpu.make_async_copy(
                    table_hbm.at[pl.ds(0, tail_b)],
                    bufs.at[kb, pl.ds(row + n_tokens + tail_a, tail_b)],
                    gsem.at[kb],
                ).wait()

        def wait_out(kb):
            pltpu.make_async_copy(
                bufs.at[kb, pl.ds(0, _G * out_rows)],
                out_hbm.at[pl.ds(0, _G * out_rows)],
                osem.at[kb],
            ).wait()

        issue_stage(0)

        def body(s, carry):
            kb = s % _NBUF
            wait_gathers(kb)
            for g in range(_G):
                for j in range(n_pref // 16):
                    bufs[kb, g * out_rows + j // per_row, pl.ds((j % per_row) * 16, 16)] = (
                        prefv[_G * s + g, pl.ds(j * 16, 16)]
                    )
            pltpu.async_copy(
                bufs.at[kb, pl.ds(0, _G * out_rows)],
                out_hbm.at[pl.ds((b0 + _G * s) * out_rows, _G * out_rows)],
                osem.at[kb],
            )

            @pl.when(s + 1 < n_stage)
            def _():
                @pl.when(s >= _NBUF - 1)
                def _():
                    wait_out((s + 1) % _NBUF)

                issue_stage(s + 1)

            return carry

        lax.fori_loop(0, n_stage, body, 0)
        for t in range(_NBUF - 1):
            wait_out((n_stage - 1 - t) % _NBUF)

    return k(table, idxp, pref)


def kernel(tokens, table, alpha, beta):
    n_tokens = alpha.shape[0]
    batch, seq = tokens.shape
    dim = table.shape[1]
    seq_tail = seq - n_tokens
    n_el = batch * n_tokens * dim

    key = jax.random.key(42)
    ka, kb = jax.random.split(key)
    keydata = jnp.stack(
        [jax.random.key_data(ka), jax.random.key_data(kb)]
    ).astype(jnp.uint32)
    ab = jnp.stack(
        [alpha.astype(jnp.float32), beta.astype(jnp.float32)]
    ).reshape(2, 1, n_tokens * dim)
    params = jnp.broadcast_to(ab, (2, batch, n_tokens * dim)).reshape(
        2, n_el // _LANES_TC, _LANES_TC
    )
    pref = _beta_prefix(keydata, params, n_el=n_el).reshape(batch, n_tokens * dim)

    tail = tokens[:, n_tokens:]
    pad = (-seq_tail) % _CHUNK
    idxp = jnp.pad(tail, ((0, 0), (0, pad))).reshape(batch, -1, _CHUNK)
    flat = _sc_embed(
        table,
        idxp,
        pref,
        batch=batch,
        n_tokens=n_tokens,
        seq_tail=seq_tail,
        dim=dim,
    )
    return flat.reshape(batch, n_tokens + seq_tail, dim)
